# baseline (speedup 1.0000x reference)
import jax
import jax.numpy as jnp
from jax.experimental import pallas as pl
from jax.experimental.pallas import tpu as pltpu

_E = 8
_R = 16
_ER = _E * _R
_SCALING = 32.0 / 16.0
_MAX_THRESHOLD = 0.125


def _fused_kernel(x_ref, wb_ref, wext_ref, bT_ref, sel_ref, bt_ref, out_ref):
    x = x_ref[...]
    base = jnp.dot(x, wb_ref[...], preferred_element_type=jnp.float32)
    y2 = jnp.dot(x, wext_ref[...], preferred_element_type=jnp.float32)
    h = y2[:, :_ER]
    rblk = y2[:, _ER:]
    lane = jax.lax.broadcasted_iota(jnp.int32, rblk.shape, 1)
    gmask = lane < _E
    gl = jnp.where(gmask, rblk, -jnp.inf)
    m = jnp.max(gl, axis=-1, keepdims=True)
    e = jnp.where(gmask, jnp.exp(gl - m), 0.0)
    p = e / jnp.sum(e, axis=-1, keepdims=True)
    tlog = jnp.sum(jnp.where(lane == _E, rblk, 0.0), axis=-1, keepdims=True)
    th = jax.nn.sigmoid(tlog + bt_ref[0, 0]) * _MAX_THRESHOLD
    ad = p - th
    w = jnp.where(gmask & (ad >= 0.0), ad, 0.0)
    wsum = jnp.sum(w, axis=-1, keepdims=True)
    wsum = jnp.where(wsum == 0.0, 1.0, wsum)
    w = w / wsum
    wexp = jnp.dot(w, sel_ref[...], preferred_element_type=jnp.float32)
    hw = (h * wexp).astype(jnp.bfloat16)
    moe = jnp.dot(hw, bT_ref[...], preferred_element_type=jnp.float32)
    out_ref[...] = base + moe * _SCALING


def kernel(inputs, W_base, A, B, Wg, Wt, bt):
    d = inputs.shape[-1]
    out_f = W_base.shape[0]
    x = inputs.reshape(-1, d)
    t = x.shape[0]
    tm = 512

    a_all = A.reshape(_ER, d)
    pad = jnp.zeros((_ER - _E - 1, d), dtype=x.dtype)
    wext = jnp.concatenate([a_all, Wg, Wt, pad], axis=0).T.astype(jnp.bfloat16)
    wbT = W_base.T.astype(jnp.bfloat16)
    bT = jnp.transpose(B, (0, 2, 1)).reshape(_ER, out_f).astype(jnp.bfloat16)
    x = x.astype(jnp.bfloat16)
    row = jax.lax.broadcasted_iota(jnp.int32, (_ER, _ER), 0)
    col = jax.lax.broadcasted_iota(jnp.int32, (_ER, _ER), 1)
    sel = ((row == col // _R) & (row < _E)).astype(jnp.float32)
    bt2 = bt.reshape(1, 1)

    grid = (t // tm,)
    out = pl.pallas_call(
        _fused_kernel,
        grid=grid,
        in_specs=[
            pl.BlockSpec((tm, d), lambda i: (i, 0)),
            pl.BlockSpec((d, out_f), lambda i: (0, 0)),
            pl.BlockSpec((d, 2 * _ER), lambda i: (0, 0)),
            pl.BlockSpec((_ER, out_f), lambda i: (0, 0)),
            pl.BlockSpec((_ER, _ER), lambda i: (0, 0)),
            pl.BlockSpec((1, 1), lambda i: (0, 0)),
        ],
        out_specs=pl.BlockSpec((tm, out_f), lambda i: (i, 0)),
        out_shape=jax.ShapeDtypeStruct((t, out_f), inputs.dtype),
        compiler_params=pltpu.CompilerParams(
            dimension_semantics=("arbitrary",),
        ),
    )(x, wbT, wext, bT, sel, bt2)
    return out.reshape(inputs.shape[:-1] + (out_f,))


# speedup vs baseline: 1.2406x; 1.2406x over previous
import jax
import jax.numpy as jnp
from jax.experimental import pallas as pl
from jax.experimental.pallas import tpu as pltpu

_E = 8
_R = 16
_ER = _E * _R
_SCALING = 32.0 / 16.0
_MAX_THRESHOLD = 0.125


def _fused_kernel(x_ref, wb_ref, wext_ref, bT_ref, sel_ref, bt_ref, out_ref):
    x = x_ref[...].astype(wb_ref.dtype)
    base = jnp.dot(x, wb_ref[...], preferred_element_type=jnp.float32)
    y2 = jnp.dot(x, wext_ref[...], preferred_element_type=jnp.float32)
    h = y2[:, :_ER]
    rblk = y2[:, _ER:]
    lane = jax.lax.broadcasted_iota(jnp.int32, rblk.shape, 1)
    gmask = lane < _E
    gl = jnp.where(gmask, rblk, -jnp.inf)
    m = jnp.max(gl, axis=-1, keepdims=True)
    e = jnp.where(gmask, jnp.exp(gl - m), 0.0)
    p = e / jnp.sum(e, axis=-1, keepdims=True)
    tlog = jnp.sum(jnp.where(lane == _E, rblk, 0.0), axis=-1, keepdims=True)
    th = jax.nn.sigmoid(tlog + bt_ref[0, 0]) * _MAX_THRESHOLD
    ad = p - th
    w = jnp.where(gmask & (ad >= 0.0), ad, 0.0)
    wsum = jnp.sum(w, axis=-1, keepdims=True)
    wsum = jnp.where(wsum == 0.0, 1.0, wsum)
    w = w / wsum
    wexp = jnp.dot(w, sel_ref[...], preferred_element_type=jnp.float32)
    hw = (h * wexp).astype(jnp.bfloat16)
    moe = jnp.dot(hw, bT_ref[...], preferred_element_type=jnp.float32)
    out_ref[...] = base + moe * _SCALING


def kernel(inputs, W_base, A, B, Wg, Wt, bt):
    d = inputs.shape[-1]
    out_f = W_base.shape[0]
    x = inputs.reshape(-1, d)
    t = x.shape[0]
    tm = 512

    a_all = A.reshape(_ER, d)
    pad = jnp.zeros((_ER - _E - 1, d), dtype=x.dtype)
    wext = jnp.concatenate([a_all, Wg, Wt, pad], axis=0).T.astype(jnp.bfloat16)
    wbT = W_base.T.astype(jnp.bfloat16)
    bT = jnp.transpose(B, (0, 2, 1)).reshape(_ER, out_f).astype(jnp.bfloat16)
    row = jax.lax.broadcasted_iota(jnp.int32, (_ER, _ER), 0)
    col = jax.lax.broadcasted_iota(jnp.int32, (_ER, _ER), 1)
    sel = ((row == col // _R) & (row < _E)).astype(jnp.float32)
    bt2 = bt.reshape(1, 1)

    grid = (t // tm,)
    out = pl.pallas_call(
        _fused_kernel,
        grid=grid,
        in_specs=[
            pl.BlockSpec((tm, d), lambda i: (i, 0)),
            pl.BlockSpec((d, out_f), lambda i: (0, 0)),
            pl.BlockSpec((d, 2 * _ER), lambda i: (0, 0)),
            pl.BlockSpec((_ER, out_f), lambda i: (0, 0)),
            pl.BlockSpec((_ER, _ER), lambda i: (0, 0)),
            pl.BlockSpec((1, 1), lambda i: (0, 0)),
        ],
        out_specs=pl.BlockSpec((tm, out_f), lambda i: (i, 0)),
        out_shape=jax.ShapeDtypeStruct((t, out_f), inputs.dtype),
        compiler_params=pltpu.CompilerParams(
            dimension_semantics=("arbitrary",),
        ),
    )(x, wbT, wext, bT, sel, bt2)
    return out.reshape(inputs.shape[:-1] + (out_f,))


# speedup vs baseline: 1.2568x; 1.0130x over previous
import jax
import jax.numpy as jnp
from jax.experimental import pallas as pl
from jax.experimental.pallas import tpu as pltpu

_E = 8
_R = 16
_ER = _E * _R
_SCALING = 32.0 / 16.0
_MAX_THRESHOLD = 0.125


def _fused_kernel(x_ref, wb_ref, wext_ref, bT_ref, sel_ref, bt_ref, out_ref):
    x = x_ref[...].astype(wb_ref.dtype)
    dn = (((1,), (1,)), ((), ()))
    base = jax.lax.dot_general(x, wb_ref[...], dn, preferred_element_type=jnp.float32)
    y2 = jax.lax.dot_general(x, wext_ref[...], dn, preferred_element_type=jnp.float32)
    h = y2[:, :_ER]
    rblk = y2[:, _ER:]
    lane = jax.lax.broadcasted_iota(jnp.int32, rblk.shape, 1)
    gmask = lane < _E
    gl = jnp.where(gmask, rblk, -jnp.inf)
    m = jnp.max(gl, axis=-1, keepdims=True)
    e = jnp.where(gmask, jnp.exp(gl - m), 0.0)
    p = e / jnp.sum(e, axis=-1, keepdims=True)
    tlog = jnp.sum(jnp.where(lane == _E, rblk, 0.0), axis=-1, keepdims=True)
    th = jax.nn.sigmoid(tlog + bt_ref[0, 0]) * _MAX_THRESHOLD
    ad = p - th
    w = jnp.where(gmask & (ad >= 0.0), ad, 0.0)
    wsum = jnp.sum(w, axis=-1, keepdims=True)
    wsum = jnp.where(wsum == 0.0, 1.0, wsum)
    w = w / wsum
    wexp = jnp.dot(w, sel_ref[...], preferred_element_type=jnp.float32)
    hw = (h * wexp).astype(jnp.bfloat16)
    moe = jnp.dot(hw, bT_ref[...], preferred_element_type=jnp.float32)
    out_ref[...] = base + moe * _SCALING


def kernel(inputs, W_base, A, B, Wg, Wt, bt):
    d = inputs.shape[-1]
    out_f = W_base.shape[0]
    x = inputs.reshape(-1, d)
    t = x.shape[0]
    tm = 512

    a_all = A.reshape(_ER, d)
    pad = jnp.zeros((_ER - _E - 1, d), dtype=x.dtype)
    wext = jnp.concatenate([a_all, Wg, Wt, pad], axis=0).astype(jnp.bfloat16)
    wb = W_base.astype(jnp.bfloat16)
    bT = jnp.transpose(B, (0, 2, 1)).reshape(_ER, out_f).astype(jnp.bfloat16)
    row = jax.lax.broadcasted_iota(jnp.int32, (_ER, _ER), 0)
    col = jax.lax.broadcasted_iota(jnp.int32, (_ER, _ER), 1)
    sel = ((row == col // _R) & (row < _E)).astype(jnp.float32)
    bt2 = bt.reshape(1, 1)

    grid = (t // tm,)
    out = pl.pallas_call(
        _fused_kernel,
        grid=grid,
        in_specs=[
            pl.BlockSpec((tm, d), lambda i: (i, 0)),
            pl.BlockSpec((out_f, d), lambda i: (0, 0)),
            pl.BlockSpec((2 * _ER, d), lambda i: (0, 0)),
            pl.BlockSpec((_ER, out_f), lambda i: (0, 0)),
            pl.BlockSpec((_ER, _ER), lambda i: (0, 0)),
            pl.BlockSpec((1, 1), lambda i: (0, 0)),
        ],
        out_specs=pl.BlockSpec((tm, out_f), lambda i: (i, 0)),
        out_shape=jax.ShapeDtypeStruct((t, out_f), inputs.dtype),
        compiler_params=pltpu.CompilerParams(
            dimension_semantics=("arbitrary",),
        ),
    )(x, wb, wext, bT, sel, bt2)
    return out.reshape(inputs.shape[:-1] + (out_f,))


# speedup vs baseline: 1.2918x; 1.0278x over previous
import jax
import jax.numpy as jnp
from jax.experimental import pallas as pl
from jax.experimental.pallas import tpu as pltpu

_E = 8
_R = 16
_ER = _E * _R
_SCALING = 32.0 / 16.0
_MAX_THRESHOLD = 0.125


def _fused_kernel(x_ref, wb_ref, wext_ref, bT_ref, sel_ref, bt_ref, out_ref):
    x = x_ref[...].astype(wb_ref.dtype)
    dn = (((1,), (1,)), ((), ()))
    base = jax.lax.dot_general(x, wb_ref[...], dn, preferred_element_type=jnp.float32)
    y2 = jax.lax.dot_general(x, wext_ref[...], dn, preferred_element_type=jnp.float32)
    h = y2[:, :_ER]
    rblk = y2[:, _ER:]
    lane = jax.lax.broadcasted_iota(jnp.int32, rblk.shape, 1)
    gmask = lane < _E
    gl = jnp.where(gmask, rblk, -jnp.inf)
    m = jnp.max(gl, axis=-1, keepdims=True)
    e = jnp.where(gmask, jnp.exp(gl - m), 0.0)
    p = e / jnp.sum(e, axis=-1, keepdims=True)
    tlog = jnp.sum(jnp.where(lane == _E, rblk, 0.0), axis=-1, keepdims=True)
    th = jax.nn.sigmoid(tlog + bt_ref[0, 0]) * _MAX_THRESHOLD
    ad = p - th
    w = jnp.where(gmask & (ad >= 0.0), ad, 0.0)
    wsum = jnp.sum(w, axis=-1, keepdims=True)
    wsum = jnp.where(wsum == 0.0, 1.0, wsum)
    w = w / wsum
    wexp = jnp.dot(w, sel_ref[...], preferred_element_type=jnp.float32)
    hw = (h * wexp).astype(jnp.bfloat16)
    moe = jnp.dot(hw, bT_ref[...], preferred_element_type=jnp.float32)
    out_ref[...] = base + moe * _SCALING


def kernel(inputs, W_base, A, B, Wg, Wt, bt):
    d = inputs.shape[-1]
    out_f = W_base.shape[0]
    x = inputs.reshape(-1, d)
    t = x.shape[0]
    tm = 1024

    a_all = A.reshape(_ER, d)
    pad = jnp.zeros((_ER - _E - 1, d), dtype=x.dtype)
    wext = jnp.concatenate([a_all, Wg, Wt, pad], axis=0).astype(jnp.bfloat16)
    wb = W_base.astype(jnp.bfloat16)
    bT = jnp.transpose(B, (0, 2, 1)).reshape(_ER, out_f).astype(jnp.bfloat16)
    row = jax.lax.broadcasted_iota(jnp.int32, (_ER, _ER), 0)
    col = jax.lax.broadcasted_iota(jnp.int32, (_ER, _ER), 1)
    sel = ((row == col // _R) & (row < _E)).astype(jnp.float32)
    bt2 = bt.reshape(1, 1)

    grid = (t // tm,)
    out = pl.pallas_call(
        _fused_kernel,
        grid=grid,
        in_specs=[
            pl.BlockSpec((tm, d), lambda i: (i, 0)),
            pl.BlockSpec((out_f, d), lambda i: (0, 0)),
            pl.BlockSpec((2 * _ER, d), lambda i: (0, 0)),
            pl.BlockSpec((_ER, out_f), lambda i: (0, 0)),
            pl.BlockSpec((_ER, _ER), lambda i: (0, 0)),
            pl.BlockSpec((1, 1), lambda i: (0, 0)),
        ],
        out_specs=pl.BlockSpec((tm, out_f), lambda i: (i, 0)),
        out_shape=jax.ShapeDtypeStruct((t, out_f), inputs.dtype),
        compiler_params=pltpu.CompilerParams(
            dimension_semantics=("arbitrary",),
        ),
    )(x, wb, wext, bT, sel, bt2)
    return out.reshape(inputs.shape[:-1] + (out_f,))


# speedup vs baseline: 1.2949x; 1.0024x over previous
import jax
import jax.numpy as jnp
from jax.experimental import pallas as pl
from jax.experimental.pallas import tpu as pltpu

_E = 8
_R = 16
_ER = _E * _R
_SCALING = 32.0 / 16.0
_MAX_THRESHOLD = 0.125


def _fused_kernel(x_ref, wb_ref, wext_ref, bT_ref, sel_ref, bt_ref, out_ref):
    x = x_ref[...]
    dn = (((1,), (1,)), ((), ()))
    base = jax.lax.dot_general(x, wb_ref[...], dn, preferred_element_type=jnp.float32)
    y2 = jax.lax.dot_general(x, wext_ref[...], dn, preferred_element_type=jnp.float32)
    h = y2[:, :_ER]
    rblk = y2[:, _ER:]
    lane = jax.lax.broadcasted_iota(jnp.int32, rblk.shape, 1)
    gmask = lane < _E
    gl = jnp.where(gmask, rblk, -jnp.inf)
    m = jnp.max(gl, axis=-1, keepdims=True)
    e = jnp.where(gmask, jnp.exp(gl - m), 0.0)
    p = e / jnp.sum(e, axis=-1, keepdims=True)
    tlog = jnp.sum(jnp.where(lane == _E, rblk, 0.0), axis=-1, keepdims=True)
    th = jax.nn.sigmoid(tlog + bt_ref[0, 0]) * _MAX_THRESHOLD
    ad = p - th
    w = jnp.where(gmask & (ad >= 0.0), ad, 0.0)
    wsum = jnp.sum(w, axis=-1, keepdims=True)
    wsum = jnp.where(wsum == 0.0, 1.0, wsum)
    w = w / wsum
    wexp = jnp.dot(w, sel_ref[...], preferred_element_type=jnp.float32)
    hw = (h * wexp).astype(jnp.bfloat16)
    moe = jnp.dot(hw, bT_ref[...], preferred_element_type=jnp.float32)
    out_ref[...] = base + moe * _SCALING


def kernel(inputs, W_base, A, B, Wg, Wt, bt):
    d = inputs.shape[-1]
    out_f = W_base.shape[0]
    x = inputs.reshape(-1, d)
    t = x.shape[0]
    tm = 1024

    a_all = A.reshape(_ER, d)
    pad = jnp.zeros((_ER - _E - 1, d), dtype=x.dtype)
    wext = jnp.concatenate([a_all, Wg, Wt, pad], axis=0).astype(jnp.bfloat16)
    wb = W_base.astype(jnp.bfloat16)
    bT = jnp.transpose(B, (0, 2, 1)).reshape(_ER, out_f).astype(jnp.bfloat16)
    row = jax.lax.broadcasted_iota(jnp.int32, (_ER, _ER), 0)
    col = jax.lax.broadcasted_iota(jnp.int32, (_ER, _ER), 1)
    sel = ((row == col // _R) & (row < _E)).astype(jnp.float32)
    bt2 = bt.reshape(1, 1)

    grid = (t // tm,)
    out = pl.pallas_call(
        _fused_kernel,
        grid=grid,
        in_specs=[
            pl.BlockSpec((tm, d), lambda i: (i, 0)),
            pl.BlockSpec((out_f, d), lambda i: (0, 0)),
            pl.BlockSpec((2 * _ER, d), lambda i: (0, 0)),
            pl.BlockSpec((_ER, out_f), lambda i: (0, 0)),
            pl.BlockSpec((_ER, _ER), lambda i: (0, 0)),
            pl.BlockSpec((1, 1), lambda i: (0, 0)),
        ],
        out_specs=pl.BlockSpec((tm, out_f), lambda i: (i, 0)),
        out_shape=jax.ShapeDtypeStruct((t, out_f), inputs.dtype),
        compiler_params=pltpu.CompilerParams(
            dimension_semantics=("arbitrary",),
        ),
    )(x, wb, wext, bT, sel, bt2)
    return out.reshape(inputs.shape[:-1] + (out_f,))


# speedup vs baseline: 1.2950x; 1.0001x over previous
import jax
import jax.numpy as jnp
from jax.experimental import pallas as pl
from jax.experimental.pallas import tpu as pltpu

_E = 8
_R = 16
_ER = _E * _R
_SCALING = 32.0 / 16.0
_MAX_THRESHOLD = 0.125


def _fused_kernel(x_ref, wb_ref, wext_ref, bT_ref, sel_ref, bt_ref, out_ref):
    x = x_ref[...]
    dn = (((1,), (1,)), ((), ()))
    base = jax.lax.dot_general(x, wb_ref[...], dn, preferred_element_type=jnp.float32)
    y2 = jax.lax.dot_general(x, wext_ref[...], dn, preferred_element_type=jnp.float32)
    h = y2[:, :_ER]
    rblk = y2[:, _ER:]
    lane = jax.lax.broadcasted_iota(jnp.int32, rblk.shape, 1)
    gmask = lane < _E
    gl = jnp.where(gmask, rblk, -jnp.inf)
    m = jnp.max(gl, axis=-1, keepdims=True)
    e = jnp.where(gmask, jnp.exp(gl - m), 0.0)
    p = e / jnp.sum(e, axis=-1, keepdims=True)
    tlog = jnp.sum(jnp.where(lane == _E, rblk, 0.0), axis=-1, keepdims=True)
    th = jax.nn.sigmoid(tlog + bt_ref[0, 0]) * _MAX_THRESHOLD
    ad = p - th
    w = jnp.where(gmask & (ad >= 0.0), ad, 0.0)
    wsum = jnp.sum(w, axis=-1, keepdims=True)
    wsum = jnp.where(wsum == 0.0, 1.0, wsum)
    w = w / wsum
    wexp = jnp.dot(w, sel_ref[...], preferred_element_type=jnp.float32)
    hw = (h * wexp).astype(jnp.bfloat16)
    moe = jnp.dot(hw, bT_ref[...], preferred_element_type=jnp.float32)
    out_ref[...] = base + moe * _SCALING


def kernel(inputs, W_base, A, B, Wg, Wt, bt):
    d = inputs.shape[-1]
    out_f = W_base.shape[0]
    x = inputs.reshape(-1, d)
    t = x.shape[0]
    tm = 1024

    a_all = A.reshape(_ER, d)
    pad = jnp.zeros((_ER - _E - 1, d), dtype=x.dtype)
    wext = jnp.concatenate([a_all, Wg, Wt, pad], axis=0).astype(jnp.bfloat16)
    wb = W_base.astype(jnp.bfloat16)
    bT = jnp.transpose(B, (0, 2, 1)).reshape(_ER, out_f).astype(jnp.bfloat16)
    row = jax.lax.broadcasted_iota(jnp.int32, (_ER, _ER), 0)
    col = jax.lax.broadcasted_iota(jnp.int32, (_ER, _ER), 1)
    sel = ((row == col // _R) & (row < _E)).astype(jnp.float32)
    bt2 = bt.reshape(1, 1)

    grid = (t // tm,)
    out = pl.pallas_call(
        _fused_kernel,
        grid=grid,
        in_specs=[
            pl.BlockSpec((tm, d), lambda i: (i, 0)),
            pl.BlockSpec((out_f, d), lambda i: (0, 0)),
            pl.BlockSpec((2 * _ER, d), lambda i: (0, 0)),
            pl.BlockSpec((_ER, out_f), lambda i: (0, 0)),
            pl.BlockSpec((_ER, _ER), lambda i: (0, 0)),
            pl.BlockSpec((1, 1), lambda i: (0, 0)),
        ],
        out_specs=pl.BlockSpec((tm, out_f), lambda i: (i, 0)),
        out_shape=jax.ShapeDtypeStruct((t, out_f), inputs.dtype),
        compiler_params=pltpu.CompilerParams(
            dimension_semantics=("parallel",),
        ),
    )(x, wb, wext, bT, sel, bt2)
    return out.reshape(inputs.shape[:-1] + (out_f,))
